# Initial kernel scaffold; baseline (speedup 1.0000x reference)
import jax
import jax.numpy as jnp
from jax.experimental import pallas as pl


def kernel(x, table, W, b):
    raise NotImplementedError("write your pallas kernel here")


# speedup vs baseline: 13.6270x; 13.6270x over previous
import functools

import jax
import jax.numpy as jnp
from jax import lax
from jax.experimental import pallas as pl
from jax.experimental.pallas import tpu as pltpu
from jax.experimental.pallas import tpu_sc as plsc

_VOCAB = 100000
_DIM = 128
_OUT = 32
_B = 4096
_L = 200

_NC = 2
_NS = 16
_NW = _NC * _NS
_IPT = _B // _NW
_LANES = 16


def _project_body(t_ref, w_ref, o_ref):
    o_ref[...] = lax.dot_general(
        t_ref[...], w_ref[...],
        dimension_numbers=(((1,), (1,)), ((), ())),
        preferred_element_type=jnp.float32,
    )


def _project_table(table, W):
    rows_blk = 5000
    grid = _VOCAB // rows_blk
    return pl.pallas_call(
        _project_body,
        grid=(grid,),
        in_specs=[
            pl.BlockSpec((rows_blk, _DIM), lambda i: (i, 0)),
            pl.BlockSpec((_OUT, _DIM), lambda i: (0, 0)),
        ],
        out_specs=pl.BlockSpec((rows_blk, _OUT), lambda i: (i, 0)),
        out_shape=jax.ShapeDtypeStruct((_VOCAB, _OUT), jnp.float32),
    )(table, W)


def _pool_body(xr_hbm, tp_hbm, b_hbm, out_hbm,
               idx_v, buf0, buf1, acc, bias_v, sem0, sem1):
    wid = lax.axis_index("s") * _NC + lax.axis_index("c")
    pltpu.sync_copy(xr_hbm.at[wid], idx_v)
    pltpu.sync_copy(b_hbm, bias_v)

    zero = jnp.zeros((_LANES,), jnp.float32)

    def zr(r, c):
        acc[r, pl.ds(0, _LANES)] = zero
        acc[r, pl.ds(_LANES, _LANES)] = zero
        return c
    lax.fori_loop(0, _IPT, zr, 0, unroll=8)

    bufs = (buf0, buf1)
    sems = (sem0, sem1)

    pltpu.make_async_copy(tp_hbm.at[idx_v.at[0]], buf0, sem0).start()
    pltpu.make_async_copy(tp_hbm.at[idx_v.at[1]], buf1, sem1).start()

    def outer(i, c):
        l0 = i * 2
        for j in range(2):
            l = l0 + j
            buf = bufs[j]
            sem = sems[j]
            pltpu.make_async_copy(tp_hbm.at[idx_v.at[l]], buf, sem).wait()

            def accum(r, cc):
                plsc.addupdate(acc.at[r, pl.ds(0, _LANES)],
                               buf[r, pl.ds(0, _LANES)])
                plsc.addupdate(acc.at[r, pl.ds(_LANES, _LANES)],
                               buf[r, pl.ds(_LANES, _LANES)])
                return cc
            lax.fori_loop(0, _IPT, accum, 0, unroll=8)

            lnext = l + 2

            @pl.when(lnext < _L)
            def _():
                pltpu.make_async_copy(tp_hbm.at[idx_v.at[lnext]], buf, sem).start()
        return c
    lax.fori_loop(0, _L // 2, outer, 0)

    scale = jnp.float32(1.0 / _L)
    blo = bias_v[pl.ds(0, _LANES)]
    bhi = bias_v[pl.ds(_LANES, _LANES)]

    def fin(r, c):
        v0 = acc[r, pl.ds(0, _LANES)] * scale + blo
        v1 = acc[r, pl.ds(_LANES, _LANES)] * scale + bhi
        acc[r, pl.ds(0, _LANES)] = 1.0 / (1.0 + jnp.exp(-v0))
        acc[r, pl.ds(_LANES, _LANES)] = 1.0 / (1.0 + jnp.exp(-v1))
        return c
    lax.fori_loop(0, _IPT, fin, 0, unroll=4)

    pltpu.sync_copy(acc, out_hbm.at[pl.ds(wid * _IPT, _IPT), :])


@functools.partial(
    pl.kernel,
    mesh=plsc.VectorSubcoreMesh(core_axis_name="c", subcore_axis_name="s"),
    compiler_params=pltpu.CompilerParams(use_tc_tiling_on_sc=False),
    out_type=jax.ShapeDtypeStruct((_B, _OUT), jnp.float32),
    scratch_types=[
        pltpu.VMEM((_L, _IPT), jnp.int32),
        pltpu.VMEM((_IPT, _OUT), jnp.float32),
        pltpu.VMEM((_IPT, _OUT), jnp.float32),
        pltpu.VMEM((_IPT, _OUT), jnp.float32),
        pltpu.VMEM((_OUT,), jnp.float32),
        pltpu.SemaphoreType.DMA,
        pltpu.SemaphoreType.DMA,
    ],
)
def _pool(xr_hbm, tp_hbm, b_hbm, out_hbm,
          idx_v, buf0, buf1, acc, bias_v, sem0, sem1):
    _pool_body(xr_hbm, tp_hbm, b_hbm, out_hbm,
               idx_v, buf0, buf1, acc, bias_v, sem0, sem1)


def kernel(x, table, W, b):
    x = x.astype(jnp.int32)
    tp = _project_table(table, W)
    xr = x.reshape(_NW, _IPT, _L).transpose(0, 2, 1)
    return _pool(xr, tp, b)
